# baseline (speedup 1.0000x reference)
import jax
import jax.numpy as jnp
from jax import lax
from jax.experimental import pallas as pl
from jax.experimental.pallas import tpu as pltpu
from jax.experimental.pallas import tpu_sc as plsc

NUM_CORES = 2
NUM_SUBCORES = 16
NUM_WORKERS = NUM_CORES * NUM_SUBCORES
LANES = 16
BLK = 128
N_BUF = 2


def _gather_body(idx_hbm, table_hbm, out_hbm, idx_v, *scratch):
    rows = scratch[:N_BUF]
    tbuf = scratch[N_BUF : 2 * N_BUF]
    gsem = scratch[2 * N_BUF : 3 * N_BUF]
    ssem = scratch[3 * N_BUF : 4 * N_BUF]

    wid = lax.axis_index("s") * NUM_CORES + lax.axis_index("c")
    nj = idx_hbm.shape[0]
    d = table_hbm.shape[1]

    pltpu.sync_copy(idx_hbm.at[:, pl.ds(wid * BLK, BLK)], idx_v)

    iota = lax.iota(jnp.int32, LANES)
    rowidx = [c0 + iota for c0 in range(0, BLK, LANES)]
    colidx = [jnp.full((LANES,), k, jnp.int32) for k in range(d)]

    def gather_start(j, b):
        pltpu.make_async_copy(table_hbm.at[idx_v.at[j]], rows[b], gsem[b]).start()

    def gather_wait(b):
        pltpu.make_async_copy(table_hbm.at[idx_v.at[0]], rows[b], gsem[b]).wait()

    def transpose(b):
        for k in range(d):
            s, r = k // 8, k % 8
            for h in range(BLK // LANES):
                v = plsc.load_gather(rows[b], [rowidx[h], colidx[k]])
                tbuf[b][s, r, pl.ds(h * LANES, LANES)] = v

    def store_start(j, b):
        pltpu.make_async_copy(
            tbuf[b], out_hbm.at[j, :, wid, :, :], ssem[b]
        ).start()

    def store_wait(j, b):
        pltpu.make_async_copy(
            tbuf[b], out_hbm.at[j, :, wid, :, :], ssem[b]
        ).wait()

    for b in range(N_BUF):
        gather_start(b, b)

    for b in range(N_BUF):
        gather_wait(b)
        transpose(b)
        gather_start(b + N_BUF, b)
        store_start(b, b)

    def body(p, carry):
        j0 = p * N_BUF
        for b in range(N_BUF):
            j = j0 + b
            gather_wait(b)
            store_wait(j, b)
            transpose(b)
            gather_start(j + N_BUF, b)
            store_start(j, b)
        return carry

    lax.fori_loop(1, nj // N_BUF - 1, body, 0)

    for b in range(N_BUF):
        j = nj - N_BUF + b
        gather_wait(b)
        store_wait(j, b)
        transpose(b)
        store_start(j, b)
        store_wait(j, b)


@jax.jit
def _embedding_lookup(idx_t, weight):
    nj, ni = idx_t.shape
    d = weight.shape[1]
    mesh = plsc.VectorSubcoreMesh(core_axis_name="c", subcore_axis_name="s")
    scratch = [pltpu.VMEM((nj, BLK), jnp.int32)]
    scratch += [pltpu.VMEM((BLK, d), jnp.float32) for _ in range(N_BUF)]
    scratch += [pltpu.VMEM((d // 8, 8, BLK), jnp.float32) for _ in range(N_BUF)]
    scratch += [pltpu.SemaphoreType.DMA for _ in range(2 * N_BUF)]
    return pl.kernel(
        _gather_body,
        out_type=jax.ShapeDtypeStruct((nj, d // 8, ni // BLK, 8, BLK), weight.dtype),
        mesh=mesh,
        scratch_types=scratch,
        compiler_params=pltpu.CompilerParams(
            use_tc_tiling_on_sc=False, needs_layout_passes=False
        ),
    )(idx_t, weight)


def kernel(token_ids, weight):
    ni, nj = token_ids.shape
    d = weight.shape[1]
    out5 = _embedding_lookup(token_ids.T.astype(jnp.int32), weight)
    return out5.transpose(2, 4, 0, 1, 3).reshape(ni, nj, d)


# speedup vs baseline: 1.3964x; 1.3964x over previous
import jax
import jax.numpy as jnp
from jax import lax
from jax.experimental import pallas as pl
from jax.experimental.pallas import tpu as pltpu
from jax.experimental.pallas import tpu_sc as plsc

NUM_CORES = 2
NUM_SUBCORES = 16
NUM_WORKERS = NUM_CORES * NUM_SUBCORES
LANES = 16
BLK = 128
N_BUF = 2


def _gather_body(idx_hbm, table_hbm, out_hbm, idx_v, *scratch):
    rows = scratch[:N_BUF]
    tbuf = scratch[N_BUF : 2 * N_BUF]
    gsem = scratch[2 * N_BUF : 3 * N_BUF]
    ssem = scratch[3 * N_BUF : 4 * N_BUF]

    wid = lax.axis_index("s") * NUM_CORES + lax.axis_index("c")
    nj = idx_hbm.shape[0]
    d = table_hbm.shape[1]

    pltpu.sync_copy(idx_hbm.at[:, pl.ds(wid * BLK, BLK)], idx_v)

    iota = lax.iota(jnp.int32, LANES)
    czero = iota - iota
    kvec = [k0 + iota for k0 in range(0, d, LANES)]

    def gather_start(j, b):
        pltpu.make_async_copy(table_hbm.at[idx_v.at[j]], rows[b], gsem[b]).start()

    def gather_wait(b):
        pltpu.make_async_copy(table_hbm.at[idx_v.at[0]], rows[b], gsem[b]).wait()

    def transpose(b):
        @plsc.parallel_loop(0, BLK, unroll=8)
        def one(c):
            cvec = czero + c
            for t in range(d // LANES):
                v = rows[b][c, pl.ds(t * LANES, LANES)]
                plsc.store_scatter(tbuf[b], [kvec[t], cvec], v)

    def store_start(j, b):
        for s in range(d // 8):
            pltpu.make_async_copy(
                tbuf[b].at[pl.ds(s * 8, 8)], out_hbm.at[j, s, wid, :, :], ssem[b]
            ).start()

    def store_wait(j, b):
        for s in range(d // 8):
            pltpu.make_async_copy(
                tbuf[b].at[pl.ds(s * 8, 8)], out_hbm.at[j, s, wid, :, :], ssem[b]
            ).wait()

    for b in range(N_BUF):
        gather_start(b, b)

    for b in range(N_BUF):
        gather_wait(b)
        transpose(b)
        gather_start(b + N_BUF, b)
        store_start(b, b)

    def body(p, carry):
        j0 = p * N_BUF
        for b in range(N_BUF):
            j = j0 + b
            gather_wait(b)
            store_wait(j, b)
            transpose(b)
            gather_start(j + N_BUF, b)
            store_start(j, b)
        return carry

    lax.fori_loop(1, nj // N_BUF - 1, body, 0)

    for b in range(N_BUF):
        j = nj - N_BUF + b
        gather_wait(b)
        store_wait(j, b)
        transpose(b)
        store_start(j, b)
        store_wait(j, b)


@jax.jit
def _embedding_lookup(idx_t, weight):
    nj, ni = idx_t.shape
    d = weight.shape[1]
    mesh = plsc.VectorSubcoreMesh(core_axis_name="c", subcore_axis_name="s")
    scratch = [pltpu.VMEM((nj, BLK), jnp.int32)]
    scratch += [pltpu.VMEM((BLK, d), jnp.float32) for _ in range(N_BUF)]
    scratch += [pltpu.VMEM((d, BLK), jnp.float32) for _ in range(N_BUF)]
    scratch += [pltpu.SemaphoreType.DMA for _ in range(2 * N_BUF)]
    return pl.kernel(
        _gather_body,
        out_type=jax.ShapeDtypeStruct((nj, d // 8, ni // BLK, 8, BLK), weight.dtype),
        mesh=mesh,
        scratch_types=scratch,
        compiler_params=pltpu.CompilerParams(
            use_tc_tiling_on_sc=False, needs_layout_passes=False
        ),
    )(idx_t, weight)


def kernel(token_ids, weight):
    ni, nj = token_ids.shape
    d = weight.shape[1]
    out5 = _embedding_lookup(token_ids.T.astype(jnp.int32), weight)
    return out5.transpose(2, 4, 0, 1, 3).reshape(ni, nj, d)


# speedup vs baseline: 2.1539x; 1.5424x over previous
import jax
import jax.numpy as jnp
from jax import lax
from jax.experimental import pallas as pl
from jax.experimental.pallas import tpu as pltpu
from jax.experimental.pallas import tpu_sc as plsc

NUM_CORES = 2
NUM_SUBCORES = 16
NUM_WORKERS = NUM_CORES * NUM_SUBCORES
LANES = 16
BLK = 128
N_BUF = 4


def _gather_body(idx_hbm, table_hbm, out_hbm, idx_v, *scratch):
    rows = scratch[:N_BUF]
    tbuf = scratch[N_BUF : 2 * N_BUF]
    gsem = scratch[2 * N_BUF : 3 * N_BUF]
    ssem = scratch[3 * N_BUF : 4 * N_BUF]

    wid = lax.axis_index("s") * NUM_CORES + lax.axis_index("c")
    nj = idx_hbm.shape[0]
    d = table_hbm.shape[1]

    pltpu.sync_copy(idx_hbm.at[:, pl.ds(wid * BLK, BLK)], idx_v)

    iota = lax.iota(jnp.int32, LANES)
    czero = iota - iota
    kvec = [k0 + iota for k0 in range(0, d, LANES)]

    def gather_start(j, b):
        pltpu.make_async_copy(table_hbm.at[idx_v.at[j]], rows[b], gsem[b]).start()

    def gather_wait(b):
        pltpu.make_async_copy(table_hbm.at[idx_v.at[0]], rows[b], gsem[b]).wait()

    def transpose(b):
        @plsc.parallel_loop(0, BLK, unroll=16)
        def one(c):
            cvec = czero + c
            for t in range(d // LANES):
                v = rows[b][c, pl.ds(t * LANES, LANES)]
                plsc.store_scatter(tbuf[b], [kvec[t], cvec], v)

    def store_start(j, b):
        for s in range(d // 8):
            pltpu.make_async_copy(
                tbuf[b].at[pl.ds(s * 8, 8), pl.ds(0, BLK)],
                out_hbm.at[j, s, wid, :, :],
                ssem[b],
            ).start()

    def store_wait(j, b):
        for s in range(d // 8):
            pltpu.make_async_copy(
                tbuf[b].at[pl.ds(s * 8, 8), pl.ds(0, BLK)],
                out_hbm.at[j, s, wid, :, :],
                ssem[b],
            ).wait()

    for b in range(N_BUF):
        gather_start(b, b)

    for b in range(N_BUF):
        gather_wait(b)
        transpose(b)
        gather_start(b + N_BUF, b)
        store_start(b, b)

    def body(p, carry):
        j0 = p * N_BUF
        for b in range(N_BUF):
            j = j0 + b
            gather_wait(b)
            store_wait(j, b)
            transpose(b)
            gather_start(j + N_BUF, b)
            store_start(j, b)
        return carry

    lax.fori_loop(1, nj // N_BUF - 1, body, 0)

    for b in range(N_BUF):
        j = nj - N_BUF + b
        gather_wait(b)
        store_wait(j, b)
        transpose(b)
        store_start(j, b)
        store_wait(j, b)


TBLK = 256
NT_FULL = 3906


def _table_body(wt_hbm, wtail_hbm, wf_hbm, *scratch):
    bufs = scratch[:2]
    tbufs = scratch[2:4]
    lsem = scratch[4:6]
    ssem = scratch[6:8]

    wid = lax.axis_index("s") * NUM_CORES + lax.axis_index("c")
    d = wt_hbm.shape[0]

    iota = lax.iota(jnp.int32, LANES)
    kvec = [k0 + iota for k0 in range(0, d, LANES)]
    zero = jnp.full((LANES,), 0, jnp.int32)

    def load_start(m, b, w=TBLK):
        pltpu.make_async_copy(
            wt_hbm.at[:, pl.ds(m * TBLK, w)],
            bufs[b].at[:, pl.ds(0, w)],
            lsem[b],
        ).start()

    def load_wait(b, w=TBLK):
        pltpu.make_async_copy(
            wt_hbm.at[:, pl.ds(0, w)],
            bufs[b].at[:, pl.ds(0, w)],
            lsem[b],
        ).wait()

    def transpose(b, na=TBLK // 4):
        @plsc.parallel_loop(0, na, unroll=8)
        def one(a):
            base = zero + 4 * a
            evecs = [base + q for q in range(4)]
            vs = [
                plsc.load_gather(bufs[b], [kvec[h % 2], evecs[h // 2]])
                for h in range(128 // LANES)
            ]
            for h, v in enumerate(vs):
                tbufs[b][a, pl.ds(16 * h, LANES)] = v

    def store_start(m, b, nr=TBLK * 32 // 128):
        pltpu.make_async_copy(
            tbufs[b].at[pl.ds(0, nr)],
            wf_hbm.at[pl.ds(m * (TBLK * d // 128), nr)],
            ssem[b],
        ).start()

    def store_wait(b, nr=TBLK * 32 // 128):
        pltpu.make_async_copy(
            tbufs[b].at[pl.ds(0, nr)], wf_hbm.at[pl.ds(0, nr)], ssem[b]
        ).wait()

    def blk(t):
        return t * NUM_WORKERS + wid

    for b in range(2):
        load_start(blk(b), b)

    for t in range(2):
        b = t % 2
        load_wait(b)
        transpose(b)
        load_start(blk(t + 2), b)
        store_start(blk(t), b)

    def body(p, carry):
        for b in range(2):
            t = 2 * p + b
            load_wait(b)
            store_wait(b)
            transpose(b)
            pl.when(blk(t + 2) < NT_FULL)(
                lambda t=t, b=b: load_start(blk(t + 2), b)
            )
            store_start(blk(t), b)
        return carry

    lax.fori_loop(1, NT_FULL // NUM_WORKERS // 2, body, 0)

    t_last = NT_FULL // NUM_WORKERS
    b = t_last % 2

    @pl.when(blk(t_last) < NT_FULL)
    def _():
        load_wait(b)
        store_wait(b)
        transpose(b)
        store_start(blk(t_last), b)

    store_wait(b)
    store_wait(1 - b)

    @pl.when(wid == 0)
    def _():
        pltpu.sync_copy(wtail_hbm, tbufs[0].at[pl.ds(0, 16)])
        pltpu.sync_copy(
            tbufs[0].at[pl.ds(0, 16)],
            wf_hbm.at[pl.ds(NT_FULL * TBLK * d // 128, 16)],
        )


@jax.jit
def _transpose_table(wt, wtail):
    d, v = wt.shape
    mesh = plsc.VectorSubcoreMesh(core_axis_name="c", subcore_axis_name="s")
    scratch = [pltpu.VMEM((d, TBLK + 1), jnp.float32) for _ in range(2)]
    scratch += [pltpu.VMEM((TBLK * d // 128, 128), jnp.float32) for _ in range(2)]
    scratch += [pltpu.SemaphoreType.DMA for _ in range(4)]
    return pl.kernel(
        _table_body,
        out_type=jax.ShapeDtypeStruct((v * d // 128, 128), wt.dtype),
        mesh=mesh,
        scratch_types=scratch,
        compiler_params=pltpu.CompilerParams(
            use_tc_tiling_on_sc=True, needs_layout_passes=False
        ),
    )(wt, wtail)


@jax.jit
def _embedding_lookup(idx_t, weight):
    nj, ni = idx_t.shape
    d = weight.shape[1]
    mesh = plsc.VectorSubcoreMesh(core_axis_name="c", subcore_axis_name="s")
    scratch = [pltpu.VMEM((nj, BLK), jnp.int32)]
    scratch += [pltpu.VMEM((BLK, d), jnp.float32) for _ in range(N_BUF)]
    scratch += [pltpu.VMEM((d, BLK + 1), jnp.float32) for _ in range(N_BUF)]
    scratch += [pltpu.SemaphoreType.DMA for _ in range(2 * N_BUF)]
    return pl.kernel(
        _gather_body,
        out_type=jax.ShapeDtypeStruct((nj, d // 8, ni // BLK, 8, BLK), weight.dtype),
        mesh=mesh,
        scratch_types=scratch,
        compiler_params=pltpu.CompilerParams(
            use_tc_tiling_on_sc=False, needs_layout_passes=False
        ),
    )(idx_t, weight)


def kernel(token_ids, weight):
    ni, nj = token_ids.shape
    d = weight.shape[1]
    ntail = weight.shape[0] % TBLK
    wtail = weight[weight.shape[0] - ntail :].reshape(ntail * d // 128, 128)
    w_rows = _transpose_table(weight.T, wtail).reshape(weight.shape)
    out5 = _embedding_lookup(token_ids.T.astype(jnp.int32), w_rows)
    return out5.transpose(2, 4, 0, 1, 3).reshape(ni, nj, d)
